# baseline (speedup 1.0000x reference)
import jax
import jax.numpy as jnp
from jax.experimental import pallas as pl
from jax.experimental.pallas import tpu as pltpu

_CH = 2048
_RAMP = (256, 256, 512, 1024)


def _chunk_schedule(batch, seq_len):
    chunks = []
    for b in range(batch):
        lens = []
        rem = seq_len
        if b == 0 and batch > 1 and seq_len >= sum(_RAMP) + _CH:
            lens.extend(_RAMP)
            rem -= sum(_RAMP)
        tail = []
        if b == batch - 1 and batch > 1 and seq_len >= sum(_RAMP) + _CH:
            tail = list(reversed(_RAMP))
            rem -= sum(_RAMP)
        while rem > 0:
            step = min(_CH, rem)
            lens.append(step)
            rem -= step
        lens.extend(tail)
        s0 = 0
        for ln in lens:
            chunks.append((b, s0, ln))
            s0 += ln
    return chunks


def _make_kernel(chunks):
    n = len(chunks)

    def body(x_hbm, e_hbm, o_hbm, emb_vmem, x_buf, o_buf, x_sem, e_sem, o_sem):
        def x_copy(k):
            b, s0, ln = chunks[k]
            return pltpu.make_async_copy(
                x_hbm.at[b, pl.ds(s0, ln), :],
                x_buf.at[k % 2, pl.ds(0, ln), :],
                x_sem.at[k % 2],
            )

        def e_copy(k):
            _, s0, ln = chunks[k]
            return pltpu.make_async_copy(
                e_hbm.at[pl.ds(s0, ln), :],
                emb_vmem.at[pl.ds(s0, ln), :],
                e_sem.at[k % 2],
            )

        def o_copy(k):
            b, s0, ln = chunks[k]
            return pltpu.make_async_copy(
                o_buf.at[k % 2, pl.ds(0, ln), :],
                o_hbm.at[b, pl.ds(s0, ln), :],
                o_sem.at[k % 2],
            )

        def start_in(k):
            x_copy(k).start()
            if chunks[k][0] == 0:
                e_copy(k).start()

        def wait_in(k):
            x_copy(k).wait()
            if chunks[k][0] == 0:
                e_copy(k).wait()

        start_in(0)
        for k in range(n):
            if k + 1 < n:
                start_in(k + 1)
            wait_in(k)
            if k >= 2:
                o_copy(k - 2).wait()
            _, s0, ln = chunks[k]
            o_buf[k % 2, :ln, :] = x_buf[k % 2, :ln, :] + emb_vmem[s0:s0 + ln, :]
            o_copy(k).start()
        if n >= 2:
            o_copy(n - 2).wait()
        o_copy(n - 1).wait()

    return body


def kernel(x, emb_table):
    batch, seq_len, hidden = x.shape
    chunks = _chunk_schedule(batch, seq_len)
    max_ln = max(ln for _, _, ln in chunks)
    return pl.pallas_call(
        _make_kernel(chunks),
        in_specs=[
            pl.BlockSpec(memory_space=pl.ANY),
            pl.BlockSpec(memory_space=pl.ANY),
        ],
        out_specs=pl.BlockSpec(memory_space=pl.ANY),
        out_shape=jax.ShapeDtypeStruct((batch, seq_len, hidden), x.dtype),
        scratch_shapes=[
            pltpu.VMEM((seq_len, hidden), x.dtype),
            pltpu.VMEM((2, max_ln, hidden), x.dtype),
            pltpu.VMEM((2, max_ln, hidden), x.dtype),
            pltpu.SemaphoreType.DMA((2,)),
            pltpu.SemaphoreType.DMA((2,)),
            pltpu.SemaphoreType.DMA((2,)),
        ],
    )(x, emb_table[:seq_len])


# speedup vs baseline: 1.0407x; 1.0407x over previous
import jax
import jax.numpy as jnp
from jax.experimental import pallas as pl
from jax.experimental.pallas import tpu as pltpu

_CH = 1024
_DEPTH = 4
_RAMP = (256, 256, 512)


def _chunk_schedule(batch, seq_len):
    chunks = []
    for b in range(batch):
        lens = []
        rem = seq_len
        if b == 0 and batch > 1 and seq_len >= sum(_RAMP) + _CH:
            lens.extend(_RAMP)
            rem -= sum(_RAMP)
        tail = []
        if b == batch - 1 and batch > 1 and seq_len >= sum(_RAMP) + _CH:
            tail = list(reversed(_RAMP))
            rem -= sum(_RAMP)
        while rem > 0:
            step = min(_CH, rem)
            lens.append(step)
            rem -= step
        lens.extend(tail)
        s0 = 0
        for ln in lens:
            chunks.append((b, s0, ln))
            s0 += ln
    return chunks


def _make_kernel(chunks, depth):
    n = len(chunks)

    def body(x_hbm, e_hbm, o_hbm, emb_vmem, x_buf, o_buf, x_sem, e_sem, o_sem):
        def x_copy(k):
            b, s0, ln = chunks[k]
            return pltpu.make_async_copy(
                x_hbm.at[b, pl.ds(s0, ln), :],
                x_buf.at[k % depth, pl.ds(0, ln), :],
                x_sem.at[k % depth],
            )

        def e_copy(k):
            _, s0, ln = chunks[k]
            return pltpu.make_async_copy(
                e_hbm.at[pl.ds(s0, ln), :],
                emb_vmem.at[pl.ds(s0, ln), :],
                e_sem.at[k % depth],
            )

        def o_copy(k):
            b, s0, ln = chunks[k]
            return pltpu.make_async_copy(
                o_buf.at[k % depth, pl.ds(0, ln), :],
                o_hbm.at[b, pl.ds(s0, ln), :],
                o_sem.at[k % depth],
            )

        def start_in(k):
            x_copy(k).start()
            if chunks[k][0] == 0:
                e_copy(k).start()

        def wait_in(k):
            x_copy(k).wait()
            if chunks[k][0] == 0:
                e_copy(k).wait()

        for j in range(min(depth - 1, n)):
            start_in(j)
        for k in range(n):
            if k + depth - 1 < n:
                start_in(k + depth - 1)
            wait_in(k)
            if k >= depth:
                o_copy(k - depth).wait()
            _, s0, ln = chunks[k]
            o_buf[k % depth, :ln, :] = x_buf[k % depth, :ln, :] + emb_vmem[s0:s0 + ln, :]
            o_copy(k).start()
        for k in range(max(0, n - depth), n):
            o_copy(k).wait()

    return body


def kernel(x, emb_table):
    batch, seq_len, hidden = x.shape
    chunks = _chunk_schedule(batch, seq_len)
    max_ln = max(ln for _, _, ln in chunks)
    depth = min(_DEPTH, len(chunks))
    return pl.pallas_call(
        _make_kernel(chunks, depth),
        in_specs=[
            pl.BlockSpec(memory_space=pl.ANY),
            pl.BlockSpec(memory_space=pl.ANY),
        ],
        out_specs=pl.BlockSpec(memory_space=pl.ANY),
        out_shape=jax.ShapeDtypeStruct((batch, seq_len, hidden), x.dtype),
        scratch_shapes=[
            pltpu.VMEM((seq_len, hidden), x.dtype),
            pltpu.VMEM((depth, max_ln, hidden), x.dtype),
            pltpu.VMEM((depth, max_ln, hidden), x.dtype),
            pltpu.SemaphoreType.DMA((depth,)),
            pltpu.SemaphoreType.DMA((depth,)),
            pltpu.SemaphoreType.DMA((depth,)),
        ],
    )(x, emb_table[:seq_len])
